# Initial kernel scaffold; baseline (speedup 1.0000x reference)
import jax
import jax.numpy as jnp
from jax.experimental import pallas as pl


def kernel(inputs, W):
    raise NotImplementedError("write your pallas kernel here")


# speedup vs baseline: 13.6862x; 13.6862x over previous
import jax
import jax.numpy as jnp
from jax import lax
from jax.experimental import pallas as pl
from jax.experimental.pallas import tpu as pltpu

B = 128
H = 2048
K = 20
DECAY = 0.95
SUB = 16
LANE = 128
BIG = (1 << 30)
NEG_KEY = -(2 ** 31)


def _tokey(x):
    k = lax.bitcast_convert_type(x, jnp.int32)
    return jnp.where(k < 0, k ^ 0x7FFFFFFF, k)


def _body(x_ref, w_ref, out_ref, enc2_ref, filt2_ref):
    enc = lax.dot_general(
        x_ref[...], w_ref[...],
        dimension_numbers=(((1,), (1,)), ((), ())),
        preferred_element_type=jnp.float32,
    )
    enc2_ref[...] = jnp.reshape(enc, (B * SUB, LANE))

    iota2 = (lax.broadcasted_iota(jnp.int32, (SUB, LANE), 0) * LANE
             + lax.broadcasted_iota(jnp.int32, (SUB, LANE), 1))

    def step(i, inh):
        e = enc2_ref[pl.ds(i * SUB, SUB), :]
        key = _tokey(jnp.abs(e) * (1.0 - inh))

        def extract(_, st):
            mask, kk = st
            m = jnp.max(kk)
            idx = jnp.min(jnp.where(kk == m, iota2, BIG))
            sel = iota2 == idx
            return (jnp.where(sel, 1.0, mask), jnp.where(sel, NEG_KEY, kk))

        mask, _ = lax.fori_loop(
            0, K, extract,
            (jnp.zeros((SUB, LANE), jnp.float32), key))
        filt2_ref[pl.ds(i * SUB, SUB), :] = e * mask
        return inh * DECAY + mask

    lax.fori_loop(0, B, step, jnp.zeros((SUB, LANE), jnp.float32))

    fkey = _tokey(jnp.reshape(filt2_ref[...], (B, H)))
    iota_l = lax.broadcasted_iota(jnp.int32, (B, H), 1)

    def extract_rows(_, st):
        out, kk = st
        m = jnp.max(kk, axis=1, keepdims=True)
        idx = jnp.min(jnp.where(kk == m, iota_l, BIG), axis=1, keepdims=True)
        sel = iota_l == idx
        return (jnp.where(sel, 1.0, out), jnp.where(sel, NEG_KEY, kk))

    out, _ = lax.fori_loop(
        0, K, extract_rows,
        (jnp.zeros((B, H), jnp.float32), fkey))
    out_ref[...] = out


def kernel(inputs, W):
    x = inputs.reshape(inputs.shape[0], -1)
    return pl.pallas_call(
        _body,
        out_shape=jax.ShapeDtypeStruct((B, H), jnp.float32),
        scratch_shapes=[
            pltpu.VMEM((B * SUB, LANE), jnp.float32),
            pltpu.VMEM((B * SUB, LANE), jnp.float32),
        ],
    )(x, W)


# speedup vs baseline: 15.0665x; 1.1009x over previous
import jax
import jax.numpy as jnp
from jax import lax
from jax.experimental import pallas as pl
from jax.experimental.pallas import tpu as pltpu

B = 128
H = 2048
K = 20
DECAY = 0.95
SUB = 16
LANE = 128
BIG = (1 << 30)
NEG_KEY = -(2 ** 31)


def _tokey(x):
    k = lax.bitcast_convert_type(x, jnp.int32)
    return jnp.where(k < 0, k ^ 0x7FFFFFFF, k)


def _body(x_ref, w_ref, out_ref, enc2_ref, filt2_ref):
    enc = lax.dot_general(
        x_ref[...], w_ref[...],
        dimension_numbers=(((1,), (1,)), ((), ())),
        preferred_element_type=jnp.float32,
    )
    enc2_ref[...] = jnp.reshape(enc, (B * SUB, LANE))

    iota2 = (lax.broadcasted_iota(jnp.int32, (SUB, LANE), 0) * LANE
             + lax.broadcasted_iota(jnp.int32, (SUB, LANE), 1))

    def step(i, inh):
        e = enc2_ref[pl.ds(i * SUB, SUB), :]
        kk = _tokey(jnp.abs(e) * (1.0 - inh))
        mask = jnp.zeros((SUB, LANE), jnp.float32)
        for _ in range(K):
            m = jnp.max(jnp.max(kk, axis=0, keepdims=True),
                        axis=1, keepdims=True)
            idx = jnp.min(jnp.where(kk == m, iota2, BIG))
            sel = iota2 == idx
            mask = jnp.where(sel, 1.0, mask)
            kk = jnp.where(sel, NEG_KEY, kk)
        filt2_ref[pl.ds(i * SUB, SUB), :] = e * mask
        return inh * DECAY + mask

    lax.fori_loop(0, B, step, jnp.zeros((SUB, LANE), jnp.float32))

    fkey = _tokey(jnp.reshape(filt2_ref[...], (B, H)))
    iota_l = lax.broadcasted_iota(jnp.int32, (B, H), 1)

    out = jnp.zeros((B, H), jnp.float32)
    for _ in range(K):
        m = jnp.max(fkey, axis=1, keepdims=True)
        idx = jnp.min(jnp.where(fkey == m, iota_l, BIG), axis=1, keepdims=True)
        sel = iota_l == idx
        out = jnp.where(sel, 1.0, out)
        fkey = jnp.where(sel, NEG_KEY, fkey)
    out_ref[...] = out


def kernel(inputs, W):
    x = inputs.reshape(inputs.shape[0], -1)
    return pl.pallas_call(
        _body,
        out_shape=jax.ShapeDtypeStruct((B, H), jnp.float32),
        scratch_shapes=[
            pltpu.VMEM((B * SUB, LANE), jnp.float32),
            pltpu.VMEM((B * SUB, LANE), jnp.float32),
        ],
    )(x, W)


# speedup vs baseline: 16.5979x; 1.1016x over previous
import jax
import jax.numpy as jnp
from jax import lax
from jax.experimental import pallas as pl
from jax.experimental.pallas import tpu as pltpu

B = 128
H = 2048
K = 20
DECAY = 0.95
SUB = 16
LANE = 128
BIG = (1 << 30)
NEG_KEY = -(2 ** 31)


def _tokey(x):
    k = lax.bitcast_convert_type(x, jnp.int32)
    return jnp.where(k < 0, k ^ 0x7FFFFFFF, k)


def _body(x_ref, w_ref, out_ref, enc2_ref, filt2_ref):
    enc = lax.dot_general(
        x_ref[...], w_ref[...],
        dimension_numbers=(((1,), (1,)), ((), ())),
        preferred_element_type=jnp.float32,
    )
    enc2_ref[...] = jnp.reshape(enc, (B * SUB, LANE))

    iota2 = (lax.broadcasted_iota(jnp.int32, (SUB, LANE), 0) * LANE
             + lax.broadcasted_iota(jnp.int32, (SUB, LANE), 1))

    def step(i, inh):
        e = enc2_ref[pl.ds(i * SUB, SUB), :]
        kk = _tokey(jnp.abs(e) * (1.0 - inh))
        mask = jnp.zeros((SUB, LANE), jnp.float32)
        for _ in range(K):
            m = jnp.max(jnp.max(kk, axis=0, keepdims=True),
                        axis=1, keepdims=True)
            idx = jnp.min(jnp.min(jnp.where(kk == m, iota2, BIG),
                                  axis=0, keepdims=True),
                          axis=1, keepdims=True)
            sel = iota2 == idx
            mask = jnp.where(sel, 1.0, mask)
            kk = jnp.where(sel, NEG_KEY, kk)
        filt2_ref[pl.ds(i * SUB, SUB), :] = e * mask
        return inh * DECAY + mask

    lax.fori_loop(0, B, step, jnp.zeros((SUB, LANE), jnp.float32))

    fkey = _tokey(jnp.reshape(filt2_ref[...], (B, H)))
    iota_l = lax.broadcasted_iota(jnp.int32, (B, H), 1)

    out = jnp.zeros((B, H), jnp.float32)
    for _ in range(K):
        m = jnp.max(fkey, axis=1, keepdims=True)
        idx = jnp.min(jnp.where(fkey == m, iota_l, BIG), axis=1, keepdims=True)
        sel = iota_l == idx
        out = jnp.where(sel, 1.0, out)
        fkey = jnp.where(sel, NEG_KEY, fkey)
    out_ref[...] = out


def kernel(inputs, W):
    x = inputs.reshape(inputs.shape[0], -1)
    return pl.pallas_call(
        _body,
        out_shape=jax.ShapeDtypeStruct((B, H), jnp.float32),
        scratch_shapes=[
            pltpu.VMEM((B * SUB, LANE), jnp.float32),
            pltpu.VMEM((B * SUB, LANE), jnp.float32),
        ],
    )(x, W)
